# Initial kernel scaffold; baseline (speedup 1.0000x reference)
import jax
import jax.numpy as jnp
from jax.experimental import pallas as pl


def kernel(x, edge_index, Wih, Whh, bih, bhh, Wl, bl, Wr):
    raise NotImplementedError("write your pallas kernel here")


# speedup vs baseline: 13.0566x; 13.0566x over previous
import functools

import jax
import jax.numpy as jnp
from jax import lax
from jax.experimental import pallas as pl
from jax.experimental.pallas import tpu as pltpu
from jax.experimental.pallas import tpu_sc as plsc

N = 10000
DEG = 32
E = N * DEG
D = 128
NLAYERS = 4

NW = 32
ROWS_W = E // NW
CH = 80
NCH = ROWS_W // CH
K = 5
NG = NCH // K

NB = 10
BN = N // NB


@functools.cache
def _sc_gather_fn():
    mesh = plsc.VectorSubcoreMesh(core_axis_name="c", subcore_axis_name="s")
    return functools.partial(
        pl.kernel,
        mesh=mesh,
        out_type=jax.ShapeDtypeStruct((E, D), jnp.float32),
        scratch_types=[
            pltpu.VMEM((NCH, CH), jnp.int32),
            pltpu.VMEM((2, K * CH, D), jnp.float32),
            pltpu.SemaphoreType.DMA,
            pltpu.SemaphoreType.DMA,
        ],
    )(_sc_gather_body)


def _sc_gather(table, idx3):
    return _sc_gather_fn()(table, idx3)


def _sc_gather_body(table_hbm, idx_hbm, out_hbm, idx_v, rows_v, gsem, wsem):
    wid = lax.axis_index("s") * 2 + lax.axis_index("c")
    base = wid * ROWS_W
    pltpu.sync_copy(idx_hbm.at[wid], idx_v)

    def group(g, carry):
        p = lax.rem(g, 2)

        @pl.when(g >= 2)
        def _():
            pltpu.make_async_copy(
                rows_v.at[p], out_hbm.at[pl.ds(0, K * CH)], wsem
            ).wait()

        descs = [
            pltpu.async_copy(
                table_hbm.at[idx_v.at[g * K + b]],
                rows_v.at[p, pl.ds(b * CH, CH)],
                gsem,
            )
            for b in range(K)
        ]
        for d in descs:
            d.wait()
        pltpu.async_copy(
            rows_v.at[p], out_hbm.at[pl.ds(base + g * (K * CH), K * CH)], wsem
        )
        return carry

    lax.fori_loop(0, NG, group, 0)
    for _ in range(2):
        pltpu.make_async_copy(
            rows_v.at[0], out_hbm.at[pl.ds(0, K * CH)], wsem
        ).wait()


def _lstm_body(g_ref, x_ref, win_ref, b_ref, wlt_ref, wrt_ref, bl_ref,
               out_ref, h_s, c_s, *, relu, resid):
    t = pl.program_id(1)

    @pl.when(t == 0)
    def _():
        h_s[...] = jnp.zeros_like(h_s)
        c_s[...] = jnp.zeros_like(c_s)

    h = h_s[...]
    cat = jnp.concatenate([g_ref[...], h], axis=1)
    z = lax.dot_general(cat, win_ref[...], (((1,), (0,)), ((), ())),
                        preferred_element_type=jnp.float32) + b_ref[...]
    gi = 1.0 / (1.0 + jnp.exp(-z[:, :D]))
    gf = 1.0 / (1.0 + jnp.exp(-z[:, D:2 * D]))
    gg = jnp.tanh(z[:, 2 * D:3 * D])
    go = 1.0 / (1.0 + jnp.exp(-z[:, 3 * D:]))
    c = gf * c_s[...] + gi * gg
    h_s[...] = go * jnp.tanh(c)
    c_s[...] = c

    @pl.when(t == DEG - 1)
    def _():
        xb = x_ref[...]
        out = (lax.dot_general(h_s[...], wlt_ref[...], (((1,), (0,)), ((), ())),
                               preferred_element_type=jnp.float32)
               + lax.dot_general(xb, wrt_ref[...], (((1,), (0,)), ((), ())),
                                 preferred_element_type=jnp.float32)
               + bl_ref[...])
        if resid:
            out = out + xb
        if relu:
            out = jnp.maximum(out, 0.0)
        out_ref[...] = out


def _lstm_layer(g, xin, win, bsum, wlt, wrt, blv, relu, resid):
    return pl.pallas_call(
        functools.partial(_lstm_body, relu=relu, resid=resid),
        grid=(NB, DEG),
        in_specs=[
            pl.BlockSpec((BN, D), lambda nb, t: (t * NB + nb, 0)),
            pl.BlockSpec((BN, D), lambda nb, t: (nb, 0)),
            pl.BlockSpec((2 * D, 4 * D), lambda nb, t: (0, 0)),
            pl.BlockSpec((1, 4 * D), lambda nb, t: (0, 0)),
            pl.BlockSpec((D, D), lambda nb, t: (0, 0)),
            pl.BlockSpec((D, D), lambda nb, t: (0, 0)),
            pl.BlockSpec((1, D), lambda nb, t: (0, 0)),
        ],
        out_specs=pl.BlockSpec((BN, D), lambda nb, t: (nb, 0)),
        out_shape=jax.ShapeDtypeStruct((N, D), jnp.float32),
        scratch_shapes=[pltpu.VMEM((BN, D), jnp.float32),
                        pltpu.VMEM((BN, D), jnp.float32)],
        compiler_params=pltpu.CompilerParams(
            dimension_semantics=("arbitrary", "arbitrary")),
    )(g, xin, win, bsum, wlt, wrt, blv)


def kernel(x, edge_index, Wih, Whh, bih, bhh, Wl, bl, Wr):
    src = edge_index[0]
    idx3 = src.reshape(N, DEG).T.reshape(NW, NCH, CH)
    h = x
    for l in range(NLAYERS):
        win = jnp.concatenate([Wih[l].T, Whh[l].T], axis=0)
        bsum = (bih[l] + bhh[l]).reshape(1, 4 * D)
        g = _sc_gather(h, idx3)
        h = _lstm_layer(g, h, win, bsum, Wl[l].T, Wr[l].T,
                        bl[l].reshape(1, D), relu=(l < 3), resid=(l in (1, 2)))
    return h


# speedup vs baseline: 16.0543x; 1.2296x over previous
import functools

import jax
import jax.numpy as jnp
from jax import lax
from jax.experimental import pallas as pl
from jax.experimental.pallas import tpu as pltpu
from jax.experimental.pallas import tpu_sc as plsc

N = 10000
DEG = 32
E = N * DEG
D = 128
NLAYERS = 4

NW = 32
ROWS_W = E // NW
CH = 80
NCH = ROWS_W // CH
K = 5
NG = NCH // K

NB = 5
BN = N // NB


@functools.cache
def _sc_gather_fn():
    mesh = plsc.VectorSubcoreMesh(core_axis_name="c", subcore_axis_name="s")
    return functools.partial(
        pl.kernel,
        mesh=mesh,
        out_type=jax.ShapeDtypeStruct((E, D), jnp.float32),
        scratch_types=[
            pltpu.VMEM((NCH, CH), jnp.int32),
            pltpu.VMEM((2, K * CH, D), jnp.float32),
            pltpu.SemaphoreType.DMA,
            pltpu.SemaphoreType.DMA,
        ],
    )(_sc_gather_body)


def _sc_gather(table, idx3):
    return _sc_gather_fn()(table, idx3)


def _sc_gather_body(table_hbm, idx_hbm, out_hbm, idx_v, rows_v, gsem, wsem):
    wid = lax.axis_index("s") * 2 + lax.axis_index("c")
    base = wid * ROWS_W
    pltpu.sync_copy(idx_hbm.at[wid], idx_v)

    def group(g, carry):
        p = lax.rem(g, 2)

        @pl.when(g >= 2)
        def _():
            pltpu.make_async_copy(
                rows_v.at[p], out_hbm.at[pl.ds(0, K * CH)], wsem
            ).wait()

        descs = [
            pltpu.async_copy(
                table_hbm.at[idx_v.at[g * K + b]],
                rows_v.at[p, pl.ds(b * CH, CH)],
                gsem,
            )
            for b in range(K)
        ]
        for d in descs:
            d.wait()
        pltpu.async_copy(
            rows_v.at[p], out_hbm.at[pl.ds(base + g * (K * CH), K * CH)], wsem
        )
        return carry

    lax.fori_loop(0, NG, group, 0)
    for _ in range(2):
        pltpu.make_async_copy(
            rows_v.at[0], out_hbm.at[pl.ds(0, K * CH)], wsem
        ).wait()


_DOT = (((1,), (0,)), ((), ()))


def _lstm_body(g_ref, x_ref, win_ref, b_ref, wlt_ref, wrt_ref, bl_ref,
               out_ref, h_s, c_s, *, relu, resid):
    t = pl.program_id(1)

    @pl.when(t == 0)
    def _():
        h_s[...] = jnp.zeros_like(h_s)
        c_s[...] = jnp.zeros_like(c_s)

    h_bf = h_s[...].astype(jnp.bfloat16)
    cat = jnp.concatenate([g_ref[...].astype(jnp.bfloat16), h_bf],
                          axis=1)
    z = lax.dot_general(cat, win_ref[...], _DOT,
                        preferred_element_type=jnp.float32) + b_ref[...]
    gi = 1.0 / (1.0 + jnp.exp(-z[:, :D]))
    gf = 1.0 / (1.0 + jnp.exp(-z[:, D:2 * D]))
    gg = jnp.tanh(z[:, 2 * D:3 * D])
    go = 1.0 / (1.0 + jnp.exp(-z[:, 3 * D:]))
    c = gf * c_s[...] + gi * gg
    h_s[...] = go * jnp.tanh(c)
    c_s[...] = c

    @pl.when(t == DEG - 1)
    def _():
        xb = x_ref[...]
        out = (lax.dot_general(h_s[...].astype(jnp.bfloat16), wlt_ref[...],
                               _DOT, preferred_element_type=jnp.float32)
               + lax.dot_general(xb.astype(jnp.bfloat16), wrt_ref[...],
                                 _DOT, preferred_element_type=jnp.float32)
               + bl_ref[...])
        if resid:
            out = out + xb
        if relu:
            out = jnp.maximum(out, 0.0)
        out_ref[...] = out


def _lstm_layer(g, xin, win, bsum, wlt, wrt, blv, relu, resid):
    out_shape = jax.ShapeDtypeStruct((N, D), jnp.float32)
    out_specs = pl.BlockSpec((BN, D), lambda nb, t: (nb, 0))
    return pl.pallas_call(
        functools.partial(_lstm_body, relu=relu, resid=resid),
        grid=(NB, DEG),
        in_specs=[
            pl.BlockSpec((BN, D), lambda nb, t: (t * NB + nb, 0)),
            pl.BlockSpec((BN, D), lambda nb, t: (nb, 0)),
            pl.BlockSpec((2 * D, 4 * D), lambda nb, t: (0, 0)),
            pl.BlockSpec((1, 4 * D), lambda nb, t: (0, 0)),
            pl.BlockSpec((D, D), lambda nb, t: (0, 0)),
            pl.BlockSpec((D, D), lambda nb, t: (0, 0)),
            pl.BlockSpec((1, D), lambda nb, t: (0, 0)),
        ],
        out_specs=out_specs,
        out_shape=out_shape,
        scratch_shapes=[pltpu.VMEM((BN, D), jnp.float32),
                        pltpu.VMEM((BN, D), jnp.float32)],
        compiler_params=pltpu.CompilerParams(
            dimension_semantics=("arbitrary", "arbitrary")),
    )(g, xin, win, bsum, wlt, wrt, blv)


def kernel(x, edge_index, Wih, Whh, bih, bhh, Wl, bl, Wr):
    src = edge_index[0]
    idx3 = src.reshape(N, DEG).T.reshape(NW, NCH, CH)
    h = x
    for l in range(NLAYERS):
        win = jnp.concatenate([Wih[l].T, Whh[l].T],
                              axis=0).astype(jnp.bfloat16)
        bsum = (bih[l] + bhh[l]).reshape(1, 4 * D)
        g = _sc_gather(h, idx3)
        h = _lstm_layer(g, h, win, bsum,
                        Wl[l].T.astype(jnp.bfloat16),
                        Wr[l].T.astype(jnp.bfloat16),
                        bl[l].reshape(1, D),
                        relu=(l < 3), resid=(l in (1, 2)))
    return h
